# baseline (speedup 1.0000x reference)
import functools

import jax
import jax.numpy as jnp
from jax import lax
from jax.experimental import pallas as pl
from jax.experimental.pallas import tpu as pltpu
from jax.experimental.pallas import tpu_sc as plsc

N = 10000
D = 128
E = 320000
EPS = 1e-15
MAXN = 1.0 - 4e-3

_NS = 16
_CHUNK = 128
_TPT = 160
_PADE = _NS * _TPT * _CHUNK
_NACC = 10240
_RPT = _NACC // _NS
_ZCH = _RPT // _CHUNK
_U = 1
_DH = 64
_DC = 16
_NB = 4
_LAG = 2
_BLK = 512


def _make_sc_seg(nsets, count, name):
    assert not count or nsets == 1
    mesh = plsc.VectorSubcoreMesh(core_axis_name="c", subcore_axis_name="s",
                                  num_cores=2, num_subcores=_NS)
    npc = 2 * nsets + (1 if count else 0)
    out_type = []
    for _ in range(2):
        out_type += [jax.ShapeDtypeStruct((_NACC, _DH), jnp.float32)] * (
            2 * nsets)
        if count:
            out_type += [jax.ShapeDtypeStruct((_NACC, _DC), jnp.float32)]
    scratch = [
        pltpu.VMEM((_TPT, _CHUNK), jnp.int32),
        pltpu.VMEM((_TPT, _CHUNK), jnp.int32),
        pltpu.VMEM((_U, _CHUNK, _DH), jnp.float32),
        pltpu.VMEM_SHARED((_NACC, _DH), jnp.float32),
        [pltpu.SemaphoreType.DMA] * _U,
    ]
    if count:
        scratch += [
            pltpu.VMEM((_CHUNK, _DC), jnp.float32),
            pltpu.VMEM_SHARED((_NACC, _DC), jnp.float32),
            [pltpu.SemaphoreType.DMA] * _U,
        ]

    @functools.partial(
        pl.kernel, mesh=mesh, out_type=out_type, scratch_types=scratch,
        name=name,
        compiler_params=pltpu.CompilerParams(use_tc_tiling_on_sc=False))
    def k(ta0, ta1, tb0, tb1, *rest):
        idx = rest[:4 * nsets]
        p = 4 * nsets
        z64_ref = rest[p]
        p += 1
        if count:
            z16_ref = rest[p]
            o16_ref = rest[p + 1]
            p += 2
        outs = rest[p:p + 2 * npc]
        p += 2 * npc
        if count:
            src_v, dst_v, rows_v, acc, gsem, ones_v, cacc, csem = rest[p:]
        else:
            src_v, dst_v, rows_v, acc, gsem = rest[p:]
        cid = lax.axis_index("c")
        sid = lax.axis_index("s")

        if count:
            pltpu.sync_copy(o16_ref, ones_v)

        def seg_pass(table_ref, out_ref, with_count, cnt_out):
            for i in range(_ZCH):
                sl = pl.ds(sid * _RPT + i * _CHUNK, _CHUNK)
                pltpu.sync_copy(z64_ref, acc.at[sl])
                if with_count:
                    pltpu.sync_copy(z16_ref, cacc.at[sl])
            plsc.subcore_barrier()

            def outer(kk, carry):
                j0 = kk * _U
                gds = [pltpu.async_copy(table_ref.at[src_v.at[j0 + b]],
                                        rows_v.at[b], gsem[b])
                       for b in range(_U)]
                cds = []
                if with_count:
                    cds = [pltpu.async_copy(ones_v, cacc.at[dst_v.at[j0 + b]],
                                            csem[b], add=True)
                           for b in range(_U)]
                for b in range(_U):
                    gds[b].wait()
                    pltpu.sync_copy(rows_v.at[b], acc.at[dst_v.at[j0 + b]],
                                    add=True)
                for c in cds:
                    c.wait()
                return carry

            lax.fori_loop(0, _TPT // _U, outer, 0)
            plsc.subcore_barrier()
            for i in range(_ZCH):
                sl = pl.ds(sid * _RPT + i * _CHUNK, _CHUNK)
                pltpu.sync_copy(acc.at[sl], out_ref.at[sl])
                if with_count:
                    pltpu.sync_copy(cacc.at[sl], cnt_out.at[sl])
            plsc.subcore_barrier()

        def run(t0, t1, work, cnt_out):
            for s, (src_ref, dst_ref, out0, out1) in enumerate(work):
                pltpu.sync_copy(src_ref.at[sid], src_v)
                pltpu.sync_copy(dst_ref.at[sid], dst_v)
                seg_pass(t0, out0, count and s == 0, cnt_out)
                seg_pass(t1, out1, False, None)

        @pl.when(cid == 0)
        def _():
            run(ta0, ta1,
                [(idx[2 * s], idx[2 * s + 1], outs[2 * s], outs[2 * s + 1])
                 for s in range(nsets)],
                outs[npc - 1] if count else None)

        @pl.when(cid == 1)
        def _():
            run(tb0, tb1,
                [(idx[2 * nsets + 2 * s], idx[2 * nsets + 2 * s + 1],
                  outs[npc + 2 * s], outs[npc + 2 * s + 1])
                 for s in range(nsets)],
                outs[2 * npc - 1] if count else None)

    return k


_make_sc_seg = functools.lru_cache(maxsize=None)(_make_sc_seg)


def _prep_edges(ei):
    pad = _PADE - E
    src = jnp.concatenate([ei[0], jnp.zeros((pad,), jnp.int32)])
    dst = jnp.concatenate([ei[1], jnp.full((pad,), N, jnp.int32)])
    return (src.reshape(_NS, _TPT, _CHUNK), dst.reshape(_NS, _TPT, _CHUNK))


def _artanh(v):
    v = jnp.clip(v, -1.0 + 1e-7, 1.0 - 1e-7)
    return 0.5 * jnp.log((1.0 + v) / (1.0 - v))


def _rownorm(v):
    return jnp.clip(jnp.sqrt(jnp.sum(v * v, axis=-1, keepdims=True)), EPS,
                    None)


def _proj(v):
    n = _rownorm(v)
    return jnp.where(n > MAXN, v / n * MAXN, v)


def _hyp_layer(u, w, b):
    xn = _rownorm(u)
    mx = lax.dot_general(u, w, (((1,), (1,)), ((), ())),
                         preferred_element_type=jnp.float32)
    mxn = _rownorm(mx)
    res = jnp.tanh(mxn / xn * _artanh(xn)) * mx / mxn
    zero = jnp.max(jnp.abs(mx), axis=-1, keepdims=True) == 0.0
    mv = _proj(jnp.where(zero, jnp.zeros_like(res), res))
    bn = _rownorm(b)
    hb = _proj(jnp.tanh(bn) * b / bn)
    x2 = jnp.sum(mv * mv, axis=-1, keepdims=True)
    y2 = jnp.sum(hb * hb, axis=-1, keepdims=True)
    xy = jnp.sum(mv * hb, axis=-1, keepdims=True)
    num = (1.0 + 2.0 * xy + y2) * mv + (1.0 - x2) * hb
    den = 1.0 + 2.0 * xy + x2 * y2
    h = _proj(num / jnp.clip(den, EPS, None))
    pn = _rownorm(h)
    xt = jax.nn.relu(h / pn * _artanh(pn))
    un = _rownorm(xt)
    return _proj(jnp.tanh(un) * xt / un)


def _tc1_body(p0, p1, cp, n0, n1, cn, xr, w1b, b1b, w1h, b1h,
              hb0_o, hb1_o, hn0_o, hn1_o):
    x = xr[...]
    pos1 = jnp.concatenate([p0[...], p1[...]], 1) / jnp.clip(cp[...], 1.0,
                                                             None)
    neg1 = jnp.concatenate([n0[...], n1[...]], 1) / jnp.clip(cn[...], 1.0,
                                                             None)
    hb = _hyp_layer(jnp.concatenate([pos1, x], 1), w1b[...], b1b[...])
    hn = _hyp_layer(jnp.concatenate([neg1, x], 1), w1h[...], b1h[...])
    hb0_o[...] = hb[:, :_DH]
    hb1_o[...] = hb[:, _DH:]
    hn0_o[...] = hn[:, :_DH]
    hn1_o[...] = hn[:, _DH:]


def _tc2_body(pb0, pb1, ph0, ph1, nb0, nb1, nh0, nh1, cp, cn,
              hb0, hb1, hn0, hn1, xr,
              w2b, b2b, w2h, b2h, w3b, b3b, w3h, b3h, out_o):
    x = xr[...]
    icp = 1.0 / jnp.clip(cp[...], 1.0, None)
    icn = 1.0 / jnp.clip(cn[...], 1.0, None)
    pos2 = jnp.concatenate([pb0[...], pb1[...], ph0[...], ph1[...]], 1) * icp
    neg2 = jnp.concatenate([nb0[...], nb1[...], nh0[...], nh1[...]], 1) * icn
    u_b2 = jnp.concatenate([pos2, hb0[...], hb1[...]], 1)
    u_n2 = jnp.concatenate([neg2, hn0[...], hn1[...]], 1)
    h_b2 = _hyp_layer(u_b2, w2b[...], b2b[...])
    h_n2 = _hyp_layer(u_n2, w2h[...], b2h[...])
    h_b3 = _hyp_layer(h_b2, w3b[...], b3b[...])
    h_n3 = _hyp_layer(h_n2, w3h[...], b3h[...])
    out_o[...] = jnp.concatenate([h_b3, h_n3, x], 1)


def _row_spec(cols):
    return pl.BlockSpec((_BLK, cols), lambda i: (i, 0))


def _full_spec(shape):
    return pl.BlockSpec(shape, lambda i: tuple(0 for _ in shape))


def _tc1(p0, p1, cp, n0, n1, cn, xp, w1b, b1b, w1h, b1h):
    grid = (_NACC // _BLK,)
    return pl.pallas_call(
        _tc1_body,
        grid=grid,
        in_specs=[_row_spec(_DH), _row_spec(_DH), _row_spec(1),
                  _row_spec(_DH), _row_spec(_DH), _row_spec(1),
                  _row_spec(D), _full_spec(w1b.shape), _full_spec((1, D)),
                  _full_spec(w1h.shape), _full_spec((1, D))],
        out_specs=[_row_spec(_DH)] * 4,
        out_shape=[jax.ShapeDtypeStruct((_NACC, _DH), jnp.float32)] * 4,
    )(p0, p1, cp, n0, n1, cn, xp, w1b, b1b, w1h, b1h)


def _tc2(sums8, cp, cn, hb0, hb1, hn0, hn1, xp,
         w2b, b2b, w2h, b2h, w3b, b3b, w3h, b3h):
    grid = (_NACC // _BLK,)
    return pl.pallas_call(
        _tc2_body,
        grid=grid,
        in_specs=[_row_spec(_DH)] * 8 + [_row_spec(1), _row_spec(1)]
                 + [_row_spec(_DH)] * 4 + [_row_spec(D),
                  _full_spec(w2b.shape), _full_spec((1, 2 * D)),
                  _full_spec(w2h.shape), _full_spec((1, 2 * D)),
                  _full_spec(w3b.shape), _full_spec((1, D)),
                  _full_spec(w3h.shape), _full_spec((1, D))],
        out_specs=[_row_spec(3 * D)],
        out_shape=[jax.ShapeDtypeStruct((_NACC, 3 * D), jnp.float32)],
    )(*sums8, cp, cn, hb0, hb1, hn0, hn1, xp,
      w2b, b2b, w2h, b2h, w3b, b3b, w3h, b3h)


def kernel(x, W1b, b1b, W1h, b1h, W2b, b2b, W2h, b2h, W3b, b3b, W3h, b3h,
           pos_edge_index, neg_edge_index):
    xp = jnp.pad(x, ((0, _NACC - N), (0, 0)))
    x0 = x[:, :_DH]
    x1 = x[:, _DH:]

    ps, pd = _prep_edges(pos_edge_index)
    ns, nd = _prep_edges(neg_edge_index)
    z64 = jnp.zeros((_CHUNK, _DH), jnp.float32)
    z16 = jnp.zeros((_CHUNK, _DC), jnp.float32)
    o16 = jnp.ones((_CHUNK, _DC), jnp.float32)

    p0, p1, pcnt, n0, n1, ncnt = _make_sc_seg(1, True, "sc_segsum_l1")(
        x0, x1, x0, x1, ps, pd, ns, nd, z64, z16, o16)
    cntp = pcnt[:, 0:1]
    cntn = ncnt[:, 0:1]

    hb0, hb1, hn0, hn1 = _tc1(p0, p1, cntp, n0, n1, cntn, xp,
                              W1b, b1b.reshape(1, D), W1h, b1h.reshape(1, D))

    sums8 = _make_sc_seg(2, False, "sc_segsum_l2")(
        hb0, hb1, hn0, hn1, ps, pd, ns, nd, ps, pd, ns, nd, z64)
    pb0, pb1, nb0, nb1, ph0, ph1, nh0, nh1 = sums8

    (out,) = _tc2((pb0, pb1, ph0, ph1, nb0, nb1, nh0, nh1), cntp, cntn,
                  hb0, hb1, hn0, hn1, xp,
                  W2b, b2b.reshape(1, 2 * D), W2h, b2h.reshape(1, 2 * D),
                  W3b, b3b.reshape(1, D), W3h, b3h.reshape(1, D))
    return out[:N]


# speedup vs baseline: 1.4855x; 1.4855x over previous
import functools

import jax
import jax.numpy as jnp
from jax import lax
from jax.experimental import pallas as pl
from jax.experimental.pallas import tpu as pltpu
from jax.experimental.pallas import tpu_sc as plsc

N = 10000
D = 128
E = 320000
EPS = 1e-15
MAXN = 1.0 - 4e-3

_NS = 16
_CHUNK = 128
_TPT = 157
_PADE = _NS * _TPT * _CHUNK
_NACC = 10240
_RPT = _NACC // _NS
_ZCH = _RPT // _CHUNK
_U = 1
_DH = 64
_DC = 16
_NB = 4
_LAG = 2
_BLK = 512


def _make_sc_seg(nsets, count, name):
    assert not count or nsets == 1
    mesh = plsc.VectorSubcoreMesh(core_axis_name="c", subcore_axis_name="s",
                                  num_cores=2, num_subcores=_NS)
    npc = 2 * nsets + (1 if count else 0)
    out_type = []
    for _ in range(2):
        out_type += [jax.ShapeDtypeStruct((_NACC, _DH), jnp.float32)] * (
            2 * nsets)
        if count:
            out_type += [jax.ShapeDtypeStruct((_NACC, _DH), jnp.float32)]
    scratch = [
        pltpu.VMEM((_TPT, _CHUNK), jnp.int32),
        pltpu.VMEM((_TPT, _CHUNK), jnp.int32),
        pltpu.VMEM((_CHUNK, _DH), jnp.float32),
        pltpu.VMEM_SHARED((_NACC, _DH), jnp.float32),
        pltpu.SemaphoreType.DMA,
    ]

    @functools.partial(
        pl.kernel, mesh=mesh, out_type=out_type, scratch_types=scratch,
        name=name,
        compiler_params=pltpu.CompilerParams(use_tc_tiling_on_sc=False))
    def k(ta0, ta1, tb0, tb1, *rest):
        idx = rest[:4 * nsets]
        p = 4 * nsets
        z64_ref = rest[p]
        p += 1
        if count:
            o64_ref = rest[p]
            p += 1
        outs = rest[p:p + 2 * npc]
        p += 2 * npc
        src_v, dst_v, rows_v, acc, sem = rest[p:]
        cid = lax.axis_index("c")
        sid = lax.axis_index("s")

        def zero_acc():
            for i in range(_ZCH):
                sl = pl.ds(sid * _RPT + i * _CHUNK, _CHUNK)
                pltpu.sync_copy(z64_ref, acc.at[sl])
            plsc.subcore_barrier()

        def copy_out(out_ref):
            plsc.subcore_barrier()
            for i in range(_ZCH):
                sl = pl.ds(sid * _RPT + i * _CHUNK, _CHUNK)
                pltpu.sync_copy(acc.at[sl], out_ref.at[sl])
            plsc.subcore_barrier()

        def seg_pass(table_ref, out_ref):
            zero_acc()

            def body(j, carry):
                pltpu.async_copy(table_ref.at[src_v.at[j]], rows_v,
                                 sem).wait()
                pltpu.sync_copy(rows_v, acc.at[dst_v.at[j]], add=True)
                return carry

            lax.fori_loop(0, _TPT, body, 0)
            copy_out(out_ref)

        def cnt_pass(cnt_out):
            pltpu.sync_copy(o64_ref, rows_v)
            zero_acc()

            def cbody(j, carry):
                pltpu.sync_copy(rows_v, acc.at[dst_v.at[j]], add=True)
                return carry

            lax.fori_loop(0, _TPT, cbody, 0)
            copy_out(cnt_out)

        def run(t0, t1, work, cnt_out):
            for s, (src_ref, dst_ref, out0, out1) in enumerate(work):
                pltpu.sync_copy(src_ref.at[sid], src_v)
                pltpu.sync_copy(dst_ref.at[sid], dst_v)
                seg_pass(t0, out0)
                seg_pass(t1, out1)
                if count and s == 0:
                    cnt_pass(cnt_out)

        @pl.when(cid == 0)
        def _():
            run(ta0, ta1,
                [(idx[2 * s], idx[2 * s + 1], outs[2 * s], outs[2 * s + 1])
                 for s in range(nsets)],
                outs[npc - 1] if count else None)

        @pl.when(cid == 1)
        def _():
            run(tb0, tb1,
                [(idx[2 * nsets + 2 * s], idx[2 * nsets + 2 * s + 1],
                  outs[npc + 2 * s], outs[npc + 2 * s + 1])
                 for s in range(nsets)],
                outs[2 * npc - 1] if count else None)

    return k


_make_sc_seg = functools.lru_cache(maxsize=None)(_make_sc_seg)


def _prep_edges(ei):
    pad = _PADE - E
    src = jnp.concatenate([ei[0], jnp.zeros((pad,), jnp.int32)])
    dst = jnp.concatenate([ei[1], jnp.full((pad,), N, jnp.int32)])
    return (src.reshape(_NS, _TPT, _CHUNK), dst.reshape(_NS, _TPT, _CHUNK))


def _artanh(v):
    v = jnp.clip(v, -1.0 + 1e-7, 1.0 - 1e-7)
    return 0.5 * jnp.log((1.0 + v) / (1.0 - v))


def _rownorm(v):
    return jnp.clip(jnp.sqrt(jnp.sum(v * v, axis=-1, keepdims=True)), EPS,
                    None)


def _proj(v):
    n = _rownorm(v)
    return jnp.where(n > MAXN, v / n * MAXN, v)


def _hyp_layer(u, w, b):
    xn = _rownorm(u)
    mx = lax.dot_general(u, w, (((1,), (1,)), ((), ())),
                         preferred_element_type=jnp.float32)
    mxn = _rownorm(mx)
    res = jnp.tanh(mxn / xn * _artanh(xn)) * mx / mxn
    zero = jnp.max(jnp.abs(mx), axis=-1, keepdims=True) == 0.0
    mv = _proj(jnp.where(zero, jnp.zeros_like(res), res))
    bn = _rownorm(b)
    hb = _proj(jnp.tanh(bn) * b / bn)
    x2 = jnp.sum(mv * mv, axis=-1, keepdims=True)
    y2 = jnp.sum(hb * hb, axis=-1, keepdims=True)
    xy = jnp.sum(mv * hb, axis=-1, keepdims=True)
    num = (1.0 + 2.0 * xy + y2) * mv + (1.0 - x2) * hb
    den = 1.0 + 2.0 * xy + x2 * y2
    h = _proj(num / jnp.clip(den, EPS, None))
    pn = _rownorm(h)
    xt = jax.nn.relu(h / pn * _artanh(pn))
    un = _rownorm(xt)
    return _proj(jnp.tanh(un) * xt / un)


def _tc1_body(p0, p1, cp, n0, n1, cn, xr, w1b, b1b, w1h, b1h,
              hb0_o, hb1_o, hn0_o, hn1_o):
    x = xr[...]
    pos1 = jnp.concatenate([p0[...], p1[...]], 1) / jnp.clip(cp[...], 1.0,
                                                             None)
    neg1 = jnp.concatenate([n0[...], n1[...]], 1) / jnp.clip(cn[...], 1.0,
                                                             None)
    hb = _hyp_layer(jnp.concatenate([pos1, x], 1), w1b[...], b1b[...])
    hn = _hyp_layer(jnp.concatenate([neg1, x], 1), w1h[...], b1h[...])
    hb0_o[...] = hb[:, :_DH]
    hb1_o[...] = hb[:, _DH:]
    hn0_o[...] = hn[:, :_DH]
    hn1_o[...] = hn[:, _DH:]


def _tc2_body(pb0, pb1, ph0, ph1, nb0, nb1, nh0, nh1, cp, cn,
              hb0, hb1, hn0, hn1, xr,
              w2b, b2b, w2h, b2h, w3b, b3b, w3h, b3h, out_o):
    x = xr[...]
    icp = 1.0 / jnp.clip(cp[...], 1.0, None)
    icn = 1.0 / jnp.clip(cn[...], 1.0, None)
    pos2 = jnp.concatenate([pb0[...], pb1[...], ph0[...], ph1[...]], 1) * icp
    neg2 = jnp.concatenate([nb0[...], nb1[...], nh0[...], nh1[...]], 1) * icn
    u_b2 = jnp.concatenate([pos2, hb0[...], hb1[...]], 1)
    u_n2 = jnp.concatenate([neg2, hn0[...], hn1[...]], 1)
    h_b2 = _hyp_layer(u_b2, w2b[...], b2b[...])
    h_n2 = _hyp_layer(u_n2, w2h[...], b2h[...])
    h_b3 = _hyp_layer(h_b2, w3b[...], b3b[...])
    h_n3 = _hyp_layer(h_n2, w3h[...], b3h[...])
    out_o[...] = jnp.concatenate([h_b3, h_n3, x], 1)


def _row_spec(cols):
    return pl.BlockSpec((_BLK, cols), lambda i: (i, 0))


def _full_spec(shape):
    return pl.BlockSpec(shape, lambda i: tuple(0 for _ in shape))


def _tc1(p0, p1, cp, n0, n1, cn, xp, w1b, b1b, w1h, b1h):
    grid = (_NACC // _BLK,)
    return pl.pallas_call(
        _tc1_body,
        grid=grid,
        in_specs=[_row_spec(_DH), _row_spec(_DH), _row_spec(1),
                  _row_spec(_DH), _row_spec(_DH), _row_spec(1),
                  _row_spec(D), _full_spec(w1b.shape), _full_spec((1, D)),
                  _full_spec(w1h.shape), _full_spec((1, D))],
        out_specs=[_row_spec(_DH)] * 4,
        out_shape=[jax.ShapeDtypeStruct((_NACC, _DH), jnp.float32)] * 4,
    )(p0, p1, cp, n0, n1, cn, xp, w1b, b1b, w1h, b1h)


def _tc2(sums8, cp, cn, hb0, hb1, hn0, hn1, xp,
         w2b, b2b, w2h, b2h, w3b, b3b, w3h, b3h):
    grid = (_NACC // _BLK,)
    return pl.pallas_call(
        _tc2_body,
        grid=grid,
        in_specs=[_row_spec(_DH)] * 8 + [_row_spec(1), _row_spec(1)]
                 + [_row_spec(_DH)] * 4 + [_row_spec(D),
                  _full_spec(w2b.shape), _full_spec((1, 2 * D)),
                  _full_spec(w2h.shape), _full_spec((1, 2 * D)),
                  _full_spec(w3b.shape), _full_spec((1, D)),
                  _full_spec(w3h.shape), _full_spec((1, D))],
        out_specs=[_row_spec(3 * D)],
        out_shape=[jax.ShapeDtypeStruct((_NACC, 3 * D), jnp.float32)],
    )(*sums8, cp, cn, hb0, hb1, hn0, hn1, xp,
      w2b, b2b, w2h, b2h, w3b, b3b, w3h, b3h)


def kernel(x, W1b, b1b, W1h, b1h, W2b, b2b, W2h, b2h, W3b, b3b, W3h, b3h,
           pos_edge_index, neg_edge_index):
    xp = jnp.pad(x, ((0, _NACC - N), (0, 0)))
    x0 = x[:, :_DH]
    x1 = x[:, _DH:]

    ps, pd = _prep_edges(pos_edge_index)
    ns, nd = _prep_edges(neg_edge_index)
    z64 = jnp.zeros((_CHUNK, _DH), jnp.float32)
    o64 = jnp.ones((_CHUNK, _DH), jnp.float32)

    p0, p1, pcnt, n0, n1, ncnt = _make_sc_seg(1, True, "sc_segsum_l1")(
        x0, x1, x0, x1, ps, pd, ns, nd, z64, o64)
    cntp = pcnt[:, 0:1]
    cntn = ncnt[:, 0:1]

    hb0, hb1, hn0, hn1 = _tc1(p0, p1, cntp, n0, n1, cntn, xp,
                              W1b, b1b.reshape(1, D), W1h, b1h.reshape(1, D))

    sums8 = _make_sc_seg(2, False, "sc_segsum_l2")(
        hb0, hb1, hn0, hn1, ps, pd, ns, nd, ps, pd, ns, nd, z64)
    pb0, pb1, nb0, nb1, ph0, ph1, nh0, nh1 = sums8

    (out,) = _tc2((pb0, pb1, ph0, ph1, nb0, nb1, nh0, nh1), cntp, cntn,
                  hb0, hb1, hn0, hn1, xp,
                  W2b, b2b.reshape(1, 2 * D), W2h, b2h.reshape(1, 2 * D),
                  W3b, b3b.reshape(1, D), W3h, b3h.reshape(1, D))
    return out[:N]


# speedup vs baseline: 1.5744x; 1.0599x over previous
import functools

import jax
import jax.numpy as jnp
from jax import lax
from jax.experimental import pallas as pl
from jax.experimental.pallas import tpu as pltpu
from jax.experimental.pallas import tpu_sc as plsc

N = 10000
D = 128
E = 320000
EPS = 1e-15
MAXN = 1.0 - 4e-3

_NS = 16
_CHUNK = 128
_TPT = 157
_PADE = _NS * _TPT * _CHUNK
_NACC = 10240
_RPT = _NACC // _NS
_ZCH = _RPT // _CHUNK
_U = 1
_DH = 64
_DC = 16
_NB = 4
_LAG = 2
_BLK = 512


def _make_sc_seg(nsets, count, name):
    assert not count or nsets == 1
    mesh = plsc.VectorSubcoreMesh(core_axis_name="c", subcore_axis_name="s",
                                  num_cores=2, num_subcores=_NS)
    npc = 2 * nsets + (1 if count else 0)
    out_type = []
    for _ in range(2):
        out_type += [jax.ShapeDtypeStruct((_NACC, _DH), jnp.float32)] * (
            2 * nsets)
        if count:
            out_type += [jax.ShapeDtypeStruct((_NACC, _DC), jnp.float32)]
    scratch = [
        pltpu.VMEM((_TPT, _CHUNK), jnp.int32),
        pltpu.VMEM((_TPT, _CHUNK), jnp.int32),
        pltpu.VMEM((_CHUNK, _DH), jnp.float32),
        pltpu.VMEM_SHARED((_NACC, _DH), jnp.float32),
        pltpu.SemaphoreType.DMA,
    ]
    if count:
        scratch += [
            pltpu.VMEM((_CHUNK, _DC), jnp.float32),
            pltpu.VMEM_SHARED((_NACC, _DC), jnp.float32),
            pltpu.SemaphoreType.DMA,
        ]

    @functools.partial(
        pl.kernel, mesh=mesh, out_type=out_type, scratch_types=scratch,
        name=name,
        compiler_params=pltpu.CompilerParams(use_tc_tiling_on_sc=False))
    def k(ta0, ta1, tb0, tb1, *rest):
        idx = rest[:4 * nsets]
        p = 4 * nsets
        z64_ref = rest[p]
        p += 1
        if count:
            z16_ref = rest[p]
            o16_ref = rest[p + 1]
            p += 2
        outs = rest[p:p + 2 * npc]
        p += 2 * npc
        if count:
            src_v, dst_v, rows_v, acc, sem, ones_v, cacc, csem = rest[p:]
        else:
            src_v, dst_v, rows_v, acc, sem = rest[p:]
        cid = lax.axis_index("c")
        sid = lax.axis_index("s")

        if count:
            pltpu.sync_copy(o16_ref, ones_v)

        def seg_pass(table_ref, out_ref, with_count, cnt_out):
            for i in range(_ZCH):
                sl = pl.ds(sid * _RPT + i * _CHUNK, _CHUNK)
                pltpu.sync_copy(z64_ref, acc.at[sl])
                if with_count:
                    pltpu.sync_copy(z16_ref, cacc.at[sl])
            plsc.subcore_barrier()

            def body(j, carry):
                gd = pltpu.async_copy(table_ref.at[src_v.at[j]], rows_v, sem)
                if with_count:
                    cd = pltpu.async_copy(ones_v, cacc.at[dst_v.at[j]], csem,
                                          add=True)
                gd.wait()
                pltpu.sync_copy(rows_v, acc.at[dst_v.at[j]], add=True)
                if with_count:
                    cd.wait()
                return carry

            lax.fori_loop(0, _TPT, body, 0)
            plsc.subcore_barrier()
            for i in range(_ZCH):
                sl = pl.ds(sid * _RPT + i * _CHUNK, _CHUNK)
                pltpu.sync_copy(acc.at[sl], out_ref.at[sl])
                if with_count:
                    pltpu.sync_copy(cacc.at[sl], cnt_out.at[sl])
            plsc.subcore_barrier()

        def run(t0, t1, work, cnt_out):
            for s, (src_ref, dst_ref, out0, out1) in enumerate(work):
                pltpu.sync_copy(src_ref.at[sid], src_v)
                pltpu.sync_copy(dst_ref.at[sid], dst_v)
                seg_pass(t0, out0, count and s == 0, cnt_out)
                seg_pass(t1, out1, False, None)

        @pl.when(cid == 0)
        def _():
            run(ta0, ta1,
                [(idx[2 * s], idx[2 * s + 1], outs[2 * s], outs[2 * s + 1])
                 for s in range(nsets)],
                outs[npc - 1] if count else None)

        @pl.when(cid == 1)
        def _():
            run(tb0, tb1,
                [(idx[2 * nsets + 2 * s], idx[2 * nsets + 2 * s + 1],
                  outs[npc + 2 * s], outs[npc + 2 * s + 1])
                 for s in range(nsets)],
                outs[2 * npc - 1] if count else None)

    return k


_make_sc_seg = functools.lru_cache(maxsize=None)(_make_sc_seg)


def _prep_edges(ei):
    pad = _PADE - E
    src = jnp.concatenate([ei[0], jnp.zeros((pad,), jnp.int32)])
    dst = jnp.concatenate([ei[1], jnp.full((pad,), N, jnp.int32)])
    return (src.reshape(_NS, _TPT, _CHUNK), dst.reshape(_NS, _TPT, _CHUNK))


def _artanh(v):
    v = jnp.clip(v, -1.0 + 1e-7, 1.0 - 1e-7)
    return 0.5 * jnp.log((1.0 + v) / (1.0 - v))


def _rownorm(v):
    return jnp.clip(jnp.sqrt(jnp.sum(v * v, axis=-1, keepdims=True)), EPS,
                    None)


def _proj(v):
    n = _rownorm(v)
    return jnp.where(n > MAXN, v / n * MAXN, v)


def _hyp_layer(u, w, b):
    xn = _rownorm(u)
    mx = lax.dot_general(u, w, (((1,), (1,)), ((), ())),
                         preferred_element_type=jnp.float32)
    mxn = _rownorm(mx)
    res = jnp.tanh(mxn / xn * _artanh(xn)) * mx / mxn
    zero = jnp.max(jnp.abs(mx), axis=-1, keepdims=True) == 0.0
    mv = _proj(jnp.where(zero, jnp.zeros_like(res), res))
    bn = _rownorm(b)
    hb = _proj(jnp.tanh(bn) * b / bn)
    x2 = jnp.sum(mv * mv, axis=-1, keepdims=True)
    y2 = jnp.sum(hb * hb, axis=-1, keepdims=True)
    xy = jnp.sum(mv * hb, axis=-1, keepdims=True)
    num = (1.0 + 2.0 * xy + y2) * mv + (1.0 - x2) * hb
    den = 1.0 + 2.0 * xy + x2 * y2
    h = _proj(num / jnp.clip(den, EPS, None))
    pn = _rownorm(h)
    xt = jax.nn.relu(h / pn * _artanh(pn))
    un = _rownorm(xt)
    return _proj(jnp.tanh(un) * xt / un)


def _tc1_body(p0, p1, cp, n0, n1, cn, xr, w1b, b1b, w1h, b1h,
              hb0_o, hb1_o, hn0_o, hn1_o):
    x = xr[...]
    pos1 = jnp.concatenate([p0[...], p1[...]], 1) / jnp.clip(cp[...], 1.0,
                                                             None)
    neg1 = jnp.concatenate([n0[...], n1[...]], 1) / jnp.clip(cn[...], 1.0,
                                                             None)
    hb = _hyp_layer(jnp.concatenate([pos1, x], 1), w1b[...], b1b[...])
    hn = _hyp_layer(jnp.concatenate([neg1, x], 1), w1h[...], b1h[...])
    hb0_o[...] = hb[:, :_DH]
    hb1_o[...] = hb[:, _DH:]
    hn0_o[...] = hn[:, :_DH]
    hn1_o[...] = hn[:, _DH:]


def _tc2_body(pb0, pb1, ph0, ph1, nb0, nb1, nh0, nh1, cp, cn,
              hb0, hb1, hn0, hn1, xr,
              w2b, b2b, w2h, b2h, w3b, b3b, w3h, b3h, out_o):
    x = xr[...]
    icp = 1.0 / jnp.clip(cp[...], 1.0, None)
    icn = 1.0 / jnp.clip(cn[...], 1.0, None)
    pos2 = jnp.concatenate([pb0[...], pb1[...], ph0[...], ph1[...]], 1) * icp
    neg2 = jnp.concatenate([nb0[...], nb1[...], nh0[...], nh1[...]], 1) * icn
    u_b2 = jnp.concatenate([pos2, hb0[...], hb1[...]], 1)
    u_n2 = jnp.concatenate([neg2, hn0[...], hn1[...]], 1)
    h_b2 = _hyp_layer(u_b2, w2b[...], b2b[...])
    h_n2 = _hyp_layer(u_n2, w2h[...], b2h[...])
    h_b3 = _hyp_layer(h_b2, w3b[...], b3b[...])
    h_n3 = _hyp_layer(h_n2, w3h[...], b3h[...])
    out_o[...] = jnp.concatenate([h_b3, h_n3, x], 1)


def _row_spec(cols):
    return pl.BlockSpec((_BLK, cols), lambda i: (i, 0))


def _full_spec(shape):
    return pl.BlockSpec(shape, lambda i: tuple(0 for _ in shape))


def _tc1(p0, p1, cp, n0, n1, cn, xp, w1b, b1b, w1h, b1h):
    grid = (_NACC // _BLK,)
    return pl.pallas_call(
        _tc1_body,
        grid=grid,
        in_specs=[_row_spec(_DH), _row_spec(_DH), _row_spec(1),
                  _row_spec(_DH), _row_spec(_DH), _row_spec(1),
                  _row_spec(D), _full_spec(w1b.shape), _full_spec((1, D)),
                  _full_spec(w1h.shape), _full_spec((1, D))],
        out_specs=[_row_spec(_DH)] * 4,
        out_shape=[jax.ShapeDtypeStruct((_NACC, _DH), jnp.float32)] * 4,
    )(p0, p1, cp, n0, n1, cn, xp, w1b, b1b, w1h, b1h)


def _tc2(sums8, cp, cn, hb0, hb1, hn0, hn1, xp,
         w2b, b2b, w2h, b2h, w3b, b3b, w3h, b3h):
    grid = (_NACC // _BLK,)
    return pl.pallas_call(
        _tc2_body,
        grid=grid,
        in_specs=[_row_spec(_DH)] * 8 + [_row_spec(1), _row_spec(1)]
                 + [_row_spec(_DH)] * 4 + [_row_spec(D),
                  _full_spec(w2b.shape), _full_spec((1, 2 * D)),
                  _full_spec(w2h.shape), _full_spec((1, 2 * D)),
                  _full_spec(w3b.shape), _full_spec((1, D)),
                  _full_spec(w3h.shape), _full_spec((1, D))],
        out_specs=[_row_spec(3 * D)],
        out_shape=[jax.ShapeDtypeStruct((_NACC, 3 * D), jnp.float32)],
    )(*sums8, cp, cn, hb0, hb1, hn0, hn1, xp,
      w2b, b2b, w2h, b2h, w3b, b3b, w3h, b3h)


def kernel(x, W1b, b1b, W1h, b1h, W2b, b2b, W2h, b2h, W3b, b3b, W3h, b3h,
           pos_edge_index, neg_edge_index):
    xp = jnp.pad(x, ((0, _NACC - N), (0, 0)))
    x0 = x[:, :_DH]
    x1 = x[:, _DH:]

    ps, pd = _prep_edges(pos_edge_index)
    ns, nd = _prep_edges(neg_edge_index)
    z64 = jnp.zeros((_CHUNK, _DH), jnp.float32)
    z16 = jnp.zeros((_CHUNK, _DC), jnp.float32)
    o16 = jnp.ones((_CHUNK, _DC), jnp.float32)

    p0, p1, pcnt, n0, n1, ncnt = _make_sc_seg(1, True, "sc_segsum_l1")(
        x0, x1, x0, x1, ps, pd, ns, nd, z64, z16, o16)
    cntp = pcnt[:, 0:1]
    cntn = ncnt[:, 0:1]

    hb0, hb1, hn0, hn1 = _tc1(p0, p1, cntp, n0, n1, cntn, xp,
                              W1b, b1b.reshape(1, D), W1h, b1h.reshape(1, D))

    sums8 = _make_sc_seg(2, False, "sc_segsum_l2")(
        hb0, hb1, hn0, hn1, ps, pd, ns, nd, ps, pd, ns, nd, z64)
    pb0, pb1, nb0, nb1, ph0, ph1, nh0, nh1 = sums8

    (out,) = _tc2((pb0, pb1, ph0, ph1, nb0, nb1, nh0, nh1), cntp, cntn,
                  hb0, hb1, hn0, hn1, xp,
                  W2b, b2b.reshape(1, 2 * D), W2h, b2h.reshape(1, 2 * D),
                  W3b, b3b.reshape(1, D), W3h, b3h.reshape(1, D))
    return out[:N]
